# Initial kernel scaffold; baseline (speedup 1.0000x reference)
import jax
import jax.numpy as jnp
from jax.experimental import pallas as pl


def kernel(x, edge_index, batch, emb, W1, b1, W2, b2, W3, b3, P1, pb1, P2, pb2, P3, pb3):
    raise NotImplementedError("write your pallas kernel here")


# speedup vs baseline: 6.7408x; 6.7408x over previous
import functools

import jax
import jax.numpy as jnp
from jax import lax
from jax.experimental import pallas as pl
from jax.experimental.pallas import tpu as pltpu
from jax.experimental.pallas import tpu_sc as plsc

N = 50000
E = 800000
NUM_FEAT = 92
H = 64
G = 256

RB = 2048
NB = -(-N // RB)
NP = NB * RB
RB2 = 256
NB2 = NP // RB2
K = 128
NSUB = 16
NCORE = 2
EP = -(-E // (NCORE * NSUB * K)) * (NCORE * NSUB * K)
GA = EP // (NSUB * K)
GC = EP // (NCORE * NSUB * K)
HH = H // 2
NQ = 8
HQ = H // NQ
ROWS_PER_SUB = NP // NSUB

_mesh = functools.partial(
    plsc.VectorSubcoreMesh, core_axis_name="c", subcore_axis_name="s")


def _sc_pair(gpair, src3, dst3, zrow):
    def body(g_hbm, src_hbm, dst_hbm, z_hbm, out_hbm,
             src_v, dst_v, buf0, buf1, zero_v, acc, sem0, sem1):
        c = lax.axis_index("c")
        s = lax.axis_index("s")
        pltpu.sync_copy(src_hbm.at[s], src_v)
        pltpu.sync_copy(dst_hbm.at[s], dst_v)
        pltpu.sync_copy(z_hbm, zero_v)
        for k in range(ROWS_PER_SUB // K):
            pltpu.sync_copy(zero_v, acc.at[pl.ds(s * ROWS_PER_SUB + k * K, K)])
        plsc.subcore_barrier()

        gh = g_hbm.at[c]

        def gather(j, buf, sem):
            return pltpu.async_copy(gh.at[src_v.at[j]], buf, sem)

        def wait(j, buf, sem):
            pltpu.make_async_copy(gh.at[src_v.at[j]], buf, sem).wait()

        def scat(j, buf):
            pltpu.sync_copy(buf, acc.at[dst_v.at[j]], add=True)

        gather(0, buf0, sem0)

        def step(k, _):
            j0 = 2 * k
            gather(j0 + 1, buf1, sem1)
            wait(j0, buf0, sem0)
            scat(j0, buf0)
            gather(j0 + 2, buf0, sem0)
            wait(j0 + 1, buf1, sem1)
            scat(j0 + 1, buf1)
            return 0

        lax.fori_loop(0, GA // 2 - 1, step, 0)
        jlast = GA - 2
        gather(jlast + 1, buf1, sem1)
        wait(jlast, buf0, sem0)
        scat(jlast, buf0)
        wait(jlast + 1, buf1, sem1)
        scat(jlast + 1, buf1)

        plsc.subcore_barrier()
        pltpu.sync_copy(acc.at[pl.ds(s * ROWS_PER_SUB, ROWS_PER_SUB)],
                        out_hbm.at[c, pl.ds(s * ROWS_PER_SUB, ROWS_PER_SUB)])

    return pl.kernel(
        body,
        out_type=jax.ShapeDtypeStruct((NCORE, NP, HQ), jnp.float32),
        mesh=_mesh(),
        compiler_params=pltpu.CompilerParams(use_tc_tiling_on_sc=False),
        scratch_types=[
            pltpu.VMEM((GA, K), jnp.int32),
            pltpu.VMEM((GA, K), jnp.int32),
            pltpu.VMEM((K, HQ), jnp.float32),
            pltpu.VMEM((K, HQ), jnp.float32),
            pltpu.VMEM((K, HQ), jnp.float32),
            pltpu.VMEM_SHARED((NP, HQ), jnp.float32),
            pltpu.SemaphoreType.DMA,
            pltpu.SemaphoreType.DMA,
        ],
    )(gpair, src3, dst3, zrow)


def _sc_count(dst4, zrow, orow):
    def body(dst_hbm, z_hbm, o_hbm, out_hbm, dst_v, zero_v, ones_v, acc):
        c = lax.axis_index("c")
        s = lax.axis_index("s")
        pltpu.sync_copy(dst_hbm.at[c, s], dst_v)
        pltpu.sync_copy(z_hbm, zero_v)
        pltpu.sync_copy(o_hbm, ones_v)
        for k in range(ROWS_PER_SUB // K):
            pltpu.sync_copy(zero_v, acc.at[pl.ds(s * ROWS_PER_SUB + k * K, K)])
        plsc.subcore_barrier()

        def step(j, _):
            pltpu.sync_copy(ones_v, acc.at[dst_v.at[j]], add=True)
            return 0

        lax.fori_loop(0, GC, step, 0)
        plsc.subcore_barrier()
        pltpu.sync_copy(acc.at[pl.ds(s * ROWS_PER_SUB, ROWS_PER_SUB)],
                        out_hbm.at[c, pl.ds(s * ROWS_PER_SUB, ROWS_PER_SUB)])

    return pl.kernel(
        body,
        out_type=jax.ShapeDtypeStruct((NCORE, NP, HQ), jnp.float32),
        mesh=_mesh(),
        compiler_params=pltpu.CompilerParams(use_tc_tiling_on_sc=False),
        scratch_types=[
            pltpu.VMEM((GC, K), jnp.int32),
            pltpu.VMEM((K, HQ), jnp.float32),
            pltpu.VMEM((K, HQ), jnp.float32),
            pltpu.VMEM_SHARED((NP, HQ), jnp.float32),
        ],
    )(dst4, zrow, orow)


def _agg(g8, src3, dst3, zrow):
    pairs = [_sc_pair(g8[qb:qb + 2], src3, dst3, zrow) for qb in range(0, NQ, 2)]
    return jnp.concatenate(pairs, axis=0)


def _split8(u):
    return jnp.stack([u[:, q * HQ:(q + 1) * HQ] for q in range(NQ)])


def _t1_body(cnt, xb, embp, w1, g_out, dinv_out):
    deg = cnt[0, :, 0:1] + cnt[1, :, 0:1] + 1.0
    dv = lax.rsqrt(deg)
    e1 = jnp.dot(embp[...], w1[...], preferred_element_type=jnp.float32)
    xr = xb[0]
    feat = lax.broadcasted_iota(jnp.int32, (128, RB2), 0)
    oh = (xr == feat).astype(jnp.float32)
    h0w = lax.dot_general(oh, e1, (((0,), (0,)), ((), ())),
                          preferred_element_type=jnp.float32)
    g = dv * h0w
    g_out[...] = _split8(g)
    dinv_out[...] = dv


def _t2_body(acc, g, dinv, b, *rest, first, last):
    idx = 0
    h_prev = None if first else rest[idx]
    idx += 0 if first else 1
    w_next = None if last else rest[idx]
    idx += 0 if last else 1
    outs = rest[idx:]
    a = jnp.concatenate([acc[q] + g[q] for q in range(NQ)], axis=1)
    dv = dinv[...]
    hn = jnp.maximum(dv * a + b[...], 0.0)
    h = hn if first else h_prev[...] + hn
    outs[0][...] = h
    if not last:
        gn = dv * jnp.dot(h, w_next[...], preferred_element_type=jnp.float32)
        outs[1][...] = _split8(gn)


def _t3_body(h3, bb, p1, q1, p2, q2, p3, q3, out, pooled, cnt):
    i = pl.program_id(0)

    @pl.when(i == 0)
    def _():
        pooled[...] = jnp.zeros((G, H), jnp.float32)
        cnt[...] = jnp.zeros((G, 1), jnp.float32)

    base = i * RB
    lane = lax.broadcasted_iota(jnp.int32, (1, RB), 1) + base
    valid_l = lane < N
    seg = lax.broadcasted_iota(jnp.int32, (G, RB), 0)
    oht = jnp.where(valid_l, (bb[0] == seg).astype(jnp.float32), 0.0)
    sub = lax.broadcasted_iota(jnp.int32, (RB, 1), 0) + base
    hm = jnp.where(sub < N, h3[...], 0.0)
    pooled[...] += lax.dot_general(oht, hm, (((1,), (0,)), ((), ())),
                                   preferred_element_type=jnp.float32)
    cnt[...] += lax.dot_general(oht, jnp.ones((RB, 1), jnp.float32),
                                (((1,), (0,)), ((), ())),
                                preferred_element_type=jnp.float32)

    @pl.when(i == NB - 1)
    def _():
        pm = pooled[...] / jnp.maximum(cnt[...], 1.0)
        z = jnp.maximum(jnp.dot(pm, p1[...],
                                preferred_element_type=jnp.float32) + q1[...], 0.0)
        z = jnp.maximum(jnp.dot(z, p2[...],
                                preferred_element_type=jnp.float32) + q2[...], 0.0)
        out[...] = jnp.dot(z, p3[...],
                           preferred_element_type=jnp.float32) + q3[...]


def _t1(cnt, xb, embp, w1):
    return pl.pallas_call(
        _t1_body,
        grid=(NB2,),
        in_specs=[
            pl.BlockSpec((NCORE, RB2, HQ), lambda i: (0, i, 0)),
            pl.BlockSpec((1, 1, RB2), lambda i: (i, 0, 0)),
            pl.BlockSpec((128, H), lambda i: (0, 0)),
            pl.BlockSpec((H, H), lambda i: (0, 0)),
        ],
        out_specs=[
            pl.BlockSpec((NQ, RB2, HQ), lambda i: (0, i, 0)),
            pl.BlockSpec((RB2, 1), lambda i: (i, 0)),
        ],
        out_shape=[
            jax.ShapeDtypeStruct((NQ, NP, HQ), jnp.float32),
            jax.ShapeDtypeStruct((NP, 1), jnp.float32),
        ],
    )(cnt, xb, embp, w1)


def _t2(acc, g, dinv, b, h_prev=None, w_next=None):
    first = h_prev is None
    last = w_next is None
    ins = [acc, g, dinv, b]
    in_specs = [
        pl.BlockSpec((NQ, RB2, HQ), lambda i: (0, i, 0)),
        pl.BlockSpec((NQ, RB2, HQ), lambda i: (0, i, 0)),
        pl.BlockSpec((RB2, 1), lambda i: (i, 0)),
        pl.BlockSpec((1, H), lambda i: (0, 0)),
    ]
    if not first:
        ins.append(h_prev)
        in_specs.append(pl.BlockSpec((RB2, H), lambda i: (i, 0)))
    if not last:
        ins.append(w_next)
        in_specs.append(pl.BlockSpec((H, H), lambda i: (0, 0)))
    out_specs = [pl.BlockSpec((RB2, H), lambda i: (i, 0))]
    out_shape = [jax.ShapeDtypeStruct((NP, H), jnp.float32)]
    if not last:
        out_specs.append(pl.BlockSpec((NQ, RB2, HQ), lambda i: (0, i, 0)))
        out_shape.append(jax.ShapeDtypeStruct((NQ, NP, HQ), jnp.float32))
    res = pl.pallas_call(
        functools.partial(_t2_body, first=first, last=last),
        grid=(NB2,),
        in_specs=in_specs,
        out_specs=out_specs,
        out_shape=out_shape,
    )(*ins)
    return res if not last else res[0]


def _t3(h3, bb, p1, q1, p2, q2, p3, q3):
    return pl.pallas_call(
        _t3_body,
        grid=(NB,),
        in_specs=[
            pl.BlockSpec((RB, H), lambda i: (i, 0)),
            pl.BlockSpec((1, 1, RB), lambda i: (i, 0, 0)),
            pl.BlockSpec((H, HH), lambda i: (0, 0)),
            pl.BlockSpec((1, HH), lambda i: (0, 0)),
            pl.BlockSpec((HH, 16), lambda i: (0, 0)),
            pl.BlockSpec((1, 16), lambda i: (0, 0)),
            pl.BlockSpec((16, 1), lambda i: (0, 0)),
            pl.BlockSpec((1, 1), lambda i: (0, 0)),
        ],
        out_specs=pl.BlockSpec((G, 1), lambda i: (0, 0)),
        out_shape=jax.ShapeDtypeStruct((G, 1), jnp.float32),
        scratch_shapes=[
            pltpu.VMEM((G, H), jnp.float32),
            pltpu.VMEM((G, 1), jnp.float32),
        ],
        compiler_params=pltpu.CompilerParams(
            dimension_semantics=("arbitrary",)),
    )(h3, bb, p1, q1, p2, q2, p3, q3)


def kernel(x, edge_index, batch, emb, W1, b1, W2, b2, W3, b3,
           P1, pb1, P2, pb2, P3, pb3):
    x32 = x.astype(jnp.int32)
    bat = batch.astype(jnp.int32)
    src = edge_index[0].astype(jnp.int32)
    dst = edge_index[1].astype(jnp.int32)

    xb = jnp.pad(x32, (0, NP - N)).reshape(NB2, 1, RB2)
    bb = jnp.pad(bat, (0, NP - N)).reshape(NB, 1, RB)
    srcp = jnp.pad(src, (0, EP - E), constant_values=N)
    dstp = jnp.pad(dst, (0, EP - E), constant_values=N)
    src3 = srcp.reshape(NSUB, GA, K)
    dst3 = dstp.reshape(NSUB, GA, K)
    embp = jnp.pad(emb, ((0, 128 - NUM_FEAT), (0, 0)))
    zrow = jnp.zeros((K, HQ), jnp.float32)
    orow = jnp.ones((K, HQ), jnp.float32)
    dst4 = dstp.reshape(NCORE, NSUB, GC, K)

    cnt = _sc_count(dst4, zrow, orow)
    g, dinv = _t1(cnt, xb, embp, W1)

    acc = _agg(g, src3, dst3, zrow)
    h, g = _t2(acc, g, dinv, b1.reshape(1, H), w_next=W2)
    acc = _agg(g, src3, dst3, zrow)
    h, g = _t2(acc, g, dinv, b2.reshape(1, H), h_prev=h, w_next=W3)
    acc = _agg(g, src3, dst3, zrow)
    h = _t2(acc, g, dinv, b3.reshape(1, H), h_prev=h)

    return _t3(h, bb, P1, pb1.reshape(1, HH), P2, pb2.reshape(1, 16),
               P3, pb3.reshape(1, 1))


# speedup vs baseline: 8.1449x; 1.2083x over previous
import functools

import jax
import jax.numpy as jnp
from jax import lax
from jax.experimental import pallas as pl
from jax.experimental.pallas import tpu as pltpu
from jax.experimental.pallas import tpu_sc as plsc

N = 50000
E = 800000
NUM_FEAT = 92
H = 64
G = 256

RB = 2048
NB = -(-N // RB)
NP = NB * RB
RB2 = 256
NB2 = NP // RB2
K = 128
NSUB = 16
NCORE = 2
EP = -(-E // (NCORE * NSUB * K)) * (NCORE * NSUB * K)
GA = EP // (NSUB * K)
GC = EP // (NCORE * NSUB * K)
HH = H // 2
NQ = 8
HQ = H // NQ
ROWS_PER_SUB = NP // NSUB

_mesh = functools.partial(
    plsc.VectorSubcoreMesh, core_axis_name="c", subcore_axis_name="s")


def _sc_pair(gpair, src3, dst3, zrow):
    def body(g_hbm, src_hbm, dst_hbm, z_hbm, out_hbm,
             src_v, dst_v, buf0, buf1, zero_v, acc, sem0, sem1):
        c = lax.axis_index("c")
        s = lax.axis_index("s")
        pltpu.sync_copy(src_hbm.at[s], src_v)
        pltpu.sync_copy(dst_hbm.at[s], dst_v)
        pltpu.sync_copy(z_hbm, zero_v)
        for k in range(ROWS_PER_SUB // K):
            pltpu.sync_copy(zero_v, acc.at[pl.ds(s * ROWS_PER_SUB + k * K, K)])
        plsc.subcore_barrier()

        gh = g_hbm.at[c]

        def gather(j, buf, sem):
            return pltpu.async_copy(gh.at[src_v.at[j]], buf, sem)

        def wait(j, buf, sem):
            pltpu.make_async_copy(gh.at[src_v.at[j]], buf, sem).wait()

        def scat(j, buf):
            pltpu.sync_copy(buf, acc.at[dst_v.at[j]], add=True)

        gather(0, buf0, sem0)

        def step(k, _):
            j0 = 2 * k
            gather(j0 + 1, buf1, sem1)
            wait(j0, buf0, sem0)
            scat(j0, buf0)
            gather(j0 + 2, buf0, sem0)
            wait(j0 + 1, buf1, sem1)
            scat(j0 + 1, buf1)
            return 0

        lax.fori_loop(0, GA // 2 - 1, step, 0)
        jlast = GA - 2
        gather(jlast + 1, buf1, sem1)
        wait(jlast, buf0, sem0)
        scat(jlast, buf0)
        wait(jlast + 1, buf1, sem1)
        scat(jlast + 1, buf1)

        plsc.subcore_barrier()
        pltpu.sync_copy(acc.at[pl.ds(s * ROWS_PER_SUB, ROWS_PER_SUB)],
                        out_hbm.at[c, pl.ds(s * ROWS_PER_SUB, ROWS_PER_SUB)])

    return pl.kernel(
        body,
        out_type=jax.ShapeDtypeStruct((NCORE, NP, HQ), jnp.float32),
        mesh=_mesh(),
        compiler_params=pltpu.CompilerParams(use_tc_tiling_on_sc=False),
        scratch_types=[
            pltpu.VMEM((GA, K), jnp.int32),
            pltpu.VMEM((GA, K), jnp.int32),
            pltpu.VMEM((K, HQ), jnp.float32),
            pltpu.VMEM((K, HQ), jnp.float32),
            pltpu.VMEM((K, HQ), jnp.float32),
            pltpu.VMEM_SHARED((NP, HQ), jnp.float32),
            pltpu.SemaphoreType.DMA,
            pltpu.SemaphoreType.DMA,
        ],
    )(gpair, src3, dst3, zrow)


def _sc_count(dst4, zrow, orow):
    def body(dst_hbm, z_hbm, o_hbm, out_hbm, dst_v, zero_v, ones_v, acc):
        c = lax.axis_index("c")
        s = lax.axis_index("s")
        pltpu.sync_copy(dst_hbm.at[c, s], dst_v)
        pltpu.sync_copy(z_hbm, zero_v)
        pltpu.sync_copy(o_hbm, ones_v)
        for k in range(ROWS_PER_SUB // K):
            pltpu.sync_copy(zero_v, acc.at[pl.ds(s * ROWS_PER_SUB + k * K, K)])
        plsc.subcore_barrier()

        def step(j, _):
            pltpu.sync_copy(ones_v, acc.at[dst_v.at[j]], add=True)
            return 0

        lax.fori_loop(0, GC, step, 0)
        plsc.subcore_barrier()
        pltpu.sync_copy(acc.at[pl.ds(s * ROWS_PER_SUB, ROWS_PER_SUB)],
                        out_hbm.at[c, pl.ds(s * ROWS_PER_SUB, ROWS_PER_SUB)])

    return pl.kernel(
        body,
        out_type=jax.ShapeDtypeStruct((NCORE, NP, HQ), jnp.float32),
        mesh=_mesh(),
        compiler_params=pltpu.CompilerParams(use_tc_tiling_on_sc=False),
        scratch_types=[
            pltpu.VMEM((GC, K), jnp.int32),
            pltpu.VMEM((K, HQ), jnp.float32),
            pltpu.VMEM((K, HQ), jnp.float32),
            pltpu.VMEM_SHARED((NP, HQ), jnp.float32),
        ],
    )(dst4, zrow, orow)


def _agg(g4, src3, dst3, zrow):
    return [_sc_pair(gp, src3, dst3, zrow) for gp in g4]


NPAIR = NQ // 2


def _pairs(u):
    return [jnp.stack([u[:, (2 * q) * HQ:(2 * q + 1) * HQ],
                       u[:, (2 * q + 1) * HQ:(2 * q + 2) * HQ]])
            for q in range(NPAIR)]


def _t1_body(cnt, xb, embp, w1, *g_outs_dinv):
    g_outs = g_outs_dinv[:NPAIR]
    dinv_out = g_outs_dinv[NPAIR]
    deg = cnt[0, :, 0:1] + cnt[1, :, 0:1] + 1.0
    dv = lax.rsqrt(deg)
    e1 = jnp.dot(embp[...], w1[...], preferred_element_type=jnp.float32)
    xr = xb[0]
    feat = lax.broadcasted_iota(jnp.int32, (128, RB2), 0)
    oh = (xr == feat).astype(jnp.float32)
    h0w = lax.dot_general(oh, e1, (((0,), (0,)), ((), ())),
                          preferred_element_type=jnp.float32)
    g = dv * h0w
    for q, gp in enumerate(_pairs(g)):
        g_outs[q][...] = gp
    dinv_out[...] = dv


def _t2_body(*refs, first, last):
    acc4 = refs[0:NPAIR]
    g4 = refs[NPAIR:2 * NPAIR]
    dinv = refs[2 * NPAIR]
    b = refs[2 * NPAIR + 1]
    idx = 2 * NPAIR + 2
    h_prev = None if first else refs[idx]
    idx += 0 if first else 1
    w_next = None if last else refs[idx]
    idx += 0 if last else 1
    outs = refs[idx:]
    a = jnp.concatenate(
        [acc4[q][p] + g4[q][p] for q in range(NPAIR) for p in range(2)],
        axis=1)
    dv = dinv[...]
    hn = jnp.maximum(dv * a + b[...], 0.0)
    h = hn if first else h_prev[...] + hn
    outs[0][...] = h
    if not last:
        gn = dv * jnp.dot(h, w_next[...], preferred_element_type=jnp.float32)
        for q, gp in enumerate(_pairs(gn)):
            outs[1 + q][...] = gp


def _t3_body(h3, bb, p1, q1, p2, q2, p3, q3, out, pooled, cnt):
    i = pl.program_id(0)

    @pl.when(i == 0)
    def _():
        pooled[...] = jnp.zeros((G, H), jnp.float32)
        cnt[...] = jnp.zeros((G, 1), jnp.float32)

    base = i * RB
    lane = lax.broadcasted_iota(jnp.int32, (1, RB), 1) + base
    valid_l = lane < N
    seg = lax.broadcasted_iota(jnp.int32, (G, RB), 0)
    oht = jnp.where(valid_l, (bb[0] == seg).astype(jnp.float32), 0.0)
    sub = lax.broadcasted_iota(jnp.int32, (RB, 1), 0) + base
    hm = jnp.where(sub < N, h3[...], 0.0)
    pooled[...] += lax.dot_general(oht, hm, (((1,), (0,)), ((), ())),
                                   preferred_element_type=jnp.float32)
    cnt[...] += lax.dot_general(oht, jnp.ones((RB, 1), jnp.float32),
                                (((1,), (0,)), ((), ())),
                                preferred_element_type=jnp.float32)

    @pl.when(i == NB - 1)
    def _():
        pm = pooled[...] / jnp.maximum(cnt[...], 1.0)
        z = jnp.maximum(jnp.dot(pm, p1[...],
                                preferred_element_type=jnp.float32) + q1[...], 0.0)
        z = jnp.maximum(jnp.dot(z, p2[...],
                                preferred_element_type=jnp.float32) + q2[...], 0.0)
        out[...] = jnp.dot(z, p3[...],
                           preferred_element_type=jnp.float32) + q3[...]


_PAIR_SPEC = pl.BlockSpec((2, RB2, HQ), lambda i: (0, i, 0))
_PAIR_SHAPE = jax.ShapeDtypeStruct((2, NP, HQ), jnp.float32)


def _t1(cnt, xb, embp, w1):
    res = pl.pallas_call(
        _t1_body,
        grid=(NB2,),
        in_specs=[
            pl.BlockSpec((NCORE, RB2, HQ), lambda i: (0, i, 0)),
            pl.BlockSpec((1, 1, RB2), lambda i: (i, 0, 0)),
            pl.BlockSpec((128, H), lambda i: (0, 0)),
            pl.BlockSpec((H, H), lambda i: (0, 0)),
        ],
        out_specs=[_PAIR_SPEC] * NPAIR + [pl.BlockSpec((RB2, 1), lambda i: (i, 0))],
        out_shape=[_PAIR_SHAPE] * NPAIR
        + [jax.ShapeDtypeStruct((NP, 1), jnp.float32)],
    )(cnt, xb, embp, w1)
    return list(res[:NPAIR]), res[NPAIR]


def _t2(acc4, g4, dinv, b, h_prev=None, w_next=None):
    first = h_prev is None
    last = w_next is None
    ins = list(acc4) + list(g4) + [dinv, b]
    in_specs = [_PAIR_SPEC] * (2 * NPAIR) + [
        pl.BlockSpec((RB2, 1), lambda i: (i, 0)),
        pl.BlockSpec((1, H), lambda i: (0, 0)),
    ]
    if not first:
        ins.append(h_prev)
        in_specs.append(pl.BlockSpec((RB2, H), lambda i: (i, 0)))
    if not last:
        ins.append(w_next)
        in_specs.append(pl.BlockSpec((H, H), lambda i: (0, 0)))
    out_specs = [pl.BlockSpec((RB2, H), lambda i: (i, 0))]
    out_shape = [jax.ShapeDtypeStruct((NP, H), jnp.float32)]
    if not last:
        out_specs += [_PAIR_SPEC] * NPAIR
        out_shape += [_PAIR_SHAPE] * NPAIR
    res = pl.pallas_call(
        functools.partial(_t2_body, first=first, last=last),
        grid=(NB2,),
        in_specs=in_specs,
        out_specs=out_specs,
        out_shape=out_shape,
    )(*ins)
    return (res[0], list(res[1:])) if not last else res[0]


def _t3(h3, bb, p1, q1, p2, q2, p3, q3):
    return pl.pallas_call(
        _t3_body,
        grid=(NB,),
        in_specs=[
            pl.BlockSpec((RB, H), lambda i: (i, 0)),
            pl.BlockSpec((1, 1, RB), lambda i: (i, 0, 0)),
            pl.BlockSpec((H, HH), lambda i: (0, 0)),
            pl.BlockSpec((1, HH), lambda i: (0, 0)),
            pl.BlockSpec((HH, 16), lambda i: (0, 0)),
            pl.BlockSpec((1, 16), lambda i: (0, 0)),
            pl.BlockSpec((16, 1), lambda i: (0, 0)),
            pl.BlockSpec((1, 1), lambda i: (0, 0)),
        ],
        out_specs=pl.BlockSpec((G, 1), lambda i: (0, 0)),
        out_shape=jax.ShapeDtypeStruct((G, 1), jnp.float32),
        scratch_shapes=[
            pltpu.VMEM((G, H), jnp.float32),
            pltpu.VMEM((G, 1), jnp.float32),
        ],
        compiler_params=pltpu.CompilerParams(
            dimension_semantics=("arbitrary",)),
    )(h3, bb, p1, q1, p2, q2, p3, q3)


def kernel(x, edge_index, batch, emb, W1, b1, W2, b2, W3, b3,
           P1, pb1, P2, pb2, P3, pb3):
    x32 = x.astype(jnp.int32)
    bat = batch.astype(jnp.int32)
    src = edge_index[0].astype(jnp.int32)
    dst = edge_index[1].astype(jnp.int32)

    xb = jnp.pad(x32, (0, NP - N)).reshape(NB2, 1, RB2)
    bb = jnp.pad(bat, (0, NP - N)).reshape(NB, 1, RB)
    srcp = jnp.pad(src, (0, EP - E), constant_values=N)
    dstp = jnp.pad(dst, (0, EP - E), constant_values=N)
    src3 = srcp.reshape(NSUB, GA, K)
    dst3 = dstp.reshape(NSUB, GA, K)
    embp = jnp.pad(emb, ((0, 128 - NUM_FEAT), (0, 0)))
    zrow = jnp.zeros((K, HQ), jnp.float32)
    orow = jnp.ones((K, HQ), jnp.float32)
    dst4 = dstp.reshape(NCORE, NSUB, GC, K)

    cnt = _sc_count(dst4, zrow, orow)
    g4, dinv = _t1(cnt, xb, embp, W1)

    acc4 = _agg(g4, src3, dst3, zrow)
    h, g4 = _t2(acc4, g4, dinv, b1.reshape(1, H), w_next=W2)
    acc4 = _agg(g4, src3, dst3, zrow)
    h, g4 = _t2(acc4, g4, dinv, b2.reshape(1, H), h_prev=h, w_next=W3)
    acc4 = _agg(g4, src3, dst3, zrow)
    h = _t2(acc4, g4, dinv, b3.reshape(1, H), h_prev=h)

    return _t3(h, bb, P1, pb1.reshape(1, HH), P2, pb2.reshape(1, 16),
               P3, pb3.reshape(1, 1))


# speedup vs baseline: 10.2027x; 1.2527x over previous
import functools

import jax
import jax.numpy as jnp
from jax import lax
from jax.experimental import pallas as pl
from jax.experimental.pallas import tpu as pltpu
from jax.experimental.pallas import tpu_sc as plsc

N = 50000
E = 800000
NUM_FEAT = 92
H = 64
G = 256

RB = 2048
NB = -(-N // RB)
NP = NB * RB
PK = NP // 16
BR = 64
RB2 = 16 * BR
NB2 = NP // RB2
K = 128
NSUB = 16
NCORE = 2
EP = -(-E // (NCORE * NSUB * K)) * (NCORE * NSUB * K)
GA = EP // (NSUB * K)
GC = EP // (NCORE * NSUB * K)
HH = H // 2
NQ = 8
HQ = H // NQ
ROWS_PER_SUB = NP // NSUB

_mesh = functools.partial(
    plsc.VectorSubcoreMesh, core_axis_name="c", subcore_axis_name="s")


def _sc_pair(gpair, src3, dst3, zrow):
    def body(g_hbm, src_hbm, dst_hbm, z_hbm, out_hbm,
             src_v, dst_v, buf0, buf1, zero_v, acc, sem0, sem1):
        c = lax.axis_index("c")
        s = lax.axis_index("s")
        pltpu.sync_copy(src_hbm.at[s], src_v)
        pltpu.sync_copy(dst_hbm.at[s], dst_v)
        pltpu.sync_copy(z_hbm, zero_v)
        for k in range(ROWS_PER_SUB // K):
            pltpu.sync_copy(zero_v, acc.at[pl.ds(s * ROWS_PER_SUB + k * K, K)])
        plsc.subcore_barrier()

        gh = g_hbm.at[c]

        def gather(j, buf, sem):
            return pltpu.async_copy(gh.at[src_v.at[j]], buf, sem)

        def wait(j, buf, sem):
            pltpu.make_async_copy(gh.at[src_v.at[j]], buf, sem).wait()

        def scat(j, buf):
            pltpu.sync_copy(buf, acc.at[dst_v.at[j]], add=True)

        gather(0, buf0, sem0)

        def step(k, _):
            j0 = 2 * k
            gather(j0 + 1, buf1, sem1)
            wait(j0, buf0, sem0)
            scat(j0, buf0)
            gather(j0 + 2, buf0, sem0)
            wait(j0 + 1, buf1, sem1)
            scat(j0 + 1, buf1)
            return 0

        lax.fori_loop(0, GA // 2 - 1, step, 0)
        jlast = GA - 2
        gather(jlast + 1, buf1, sem1)
        wait(jlast, buf0, sem0)
        scat(jlast, buf0)
        wait(jlast + 1, buf1, sem1)
        scat(jlast + 1, buf1)

        plsc.subcore_barrier()
        pltpu.sync_copy(acc.at[pl.ds(s * ROWS_PER_SUB, ROWS_PER_SUB)],
                        out_hbm.at[c, pl.ds(s * ROWS_PER_SUB, ROWS_PER_SUB)])

    return pl.kernel(
        body,
        out_type=jax.ShapeDtypeStruct((NCORE, NP, HQ), jnp.float32),
        mesh=_mesh(),
        compiler_params=pltpu.CompilerParams(use_tc_tiling_on_sc=False),
        scratch_types=[
            pltpu.VMEM((GA, K), jnp.int32),
            pltpu.VMEM((GA, K), jnp.int32),
            pltpu.VMEM((K, HQ), jnp.float32),
            pltpu.VMEM((K, HQ), jnp.float32),
            pltpu.VMEM((K, HQ), jnp.float32),
            pltpu.VMEM_SHARED((NP, HQ), jnp.float32),
            pltpu.SemaphoreType.DMA,
            pltpu.SemaphoreType.DMA,
        ],
    )(gpair, src3, dst3, zrow)


def _sc_count(dst4, zrow, orow):
    def body(dst_hbm, z_hbm, o_hbm, out_hbm, dst_v, zero_v, ones_v, acc):
        c = lax.axis_index("c")
        s = lax.axis_index("s")
        pltpu.sync_copy(dst_hbm.at[c, s], dst_v)
        pltpu.sync_copy(z_hbm, zero_v)
        pltpu.sync_copy(o_hbm, ones_v)
        for k in range(ROWS_PER_SUB // K):
            pltpu.sync_copy(zero_v, acc.at[pl.ds(s * ROWS_PER_SUB + k * K, K)])
        plsc.subcore_barrier()

        def step(j, _):
            pltpu.sync_copy(ones_v, acc.at[dst_v.at[j]], add=True)
            return 0

        lax.fori_loop(0, GC, step, 0)
        plsc.subcore_barrier()
        pltpu.sync_copy(acc.at[pl.ds(s * ROWS_PER_SUB, ROWS_PER_SUB)],
                        out_hbm.at[c, pl.ds(s * ROWS_PER_SUB, ROWS_PER_SUB)])

    return pl.kernel(
        body,
        out_type=jax.ShapeDtypeStruct((NCORE, NP, HQ), jnp.float32),
        mesh=_mesh(),
        compiler_params=pltpu.CompilerParams(use_tc_tiling_on_sc=False),
        scratch_types=[
            pltpu.VMEM((GC, K), jnp.int32),
            pltpu.VMEM((K, HQ), jnp.float32),
            pltpu.VMEM((K, HQ), jnp.float32),
            pltpu.VMEM_SHARED((NP, HQ), jnp.float32),
        ],
    )(dst4, zrow, orow)


def _agg(g4, src3, dst3, zrow):
    return [_sc_pair(gp, src3, dst3, zrow) for gp in g4]


NPAIR = NQ // 2


def _dvp(cnt):
    return lax.rsqrt(cnt[0] + cnt[1] + 1.0)


def _t1_body(cnt, xp, embp, w1, *g_outs):
    dvp = _dvp(cnt)
    e1 = jnp.dot(embp[...], w1[...], preferred_element_type=jnp.float32)
    cols = []
    for a in range(16):
        xa = xp[:, a:a + 1]
        oh = (xa == lax.broadcasted_iota(jnp.int32, (BR, 128), 1)
              ).astype(jnp.float32)
        ra = jnp.dot(oh, e1, preferred_element_type=jnp.float32)
        cols.append(dvp[:, 8 * a:8 * a + 1] * ra)
    for q in range(NPAIR):
        g_outs[q][...] = jnp.stack(
            [jnp.concatenate(
                [cols[a][:, 8 * j:8 * j + 8] for a in range(16)], axis=1)
             for j in (2 * q, 2 * q + 1)])


def _t2_body(*refs, first, last):
    acc4 = refs[0:NPAIR]
    g4 = refs[NPAIR:2 * NPAIR]
    cnt = refs[2 * NPAIR]
    bp = refs[2 * NPAIR + 1]
    idx = 2 * NPAIR + 2
    hp4 = None if first else refs[idx:idx + NPAIR]
    idx += 0 if first else NPAIR
    wbig = None if last else refs[idx]
    idx += 0 if last else 1
    outs = refs[idx:]
    dvp = _dvp(cnt)
    slabs = []
    for q in range(NPAIR):
        hn = []
        for p in range(2):
            j = 2 * q + p
            v = jnp.maximum(dvp * (acc4[q][p] + g4[q][p]) + bp[j:j + 1, :],
                            0.0)
            if not first:
                v = v + hp4[q][p]
            hn.append(v)
        outs[q][...] = jnp.stack(hn)
        slabs.extend(hn)
    if not last:
        h_all = jnp.concatenate(slabs, axis=1)
        gn = jnp.dot(h_all, wbig[...], preferred_element_type=jnp.float32)
        for q in range(NPAIR):
            outs[NPAIR + q][...] = jnp.stack(
                [dvp * gn[:, (2 * q) * 128:(2 * q + 1) * 128],
                 dvp * gn[:, (2 * q + 1) * 128:(2 * q + 2) * 128]])


def _t3_body(h3, bb, p1, q1, p2, q2, p3, q3, out, pooled, cnt):
    i = pl.program_id(0)

    @pl.when(i == 0)
    def _():
        pooled[...] = jnp.zeros((G, H), jnp.float32)
        cnt[...] = jnp.zeros((G, 1), jnp.float32)

    base = i * RB
    lane = lax.broadcasted_iota(jnp.int32, (1, RB), 1) + base
    valid_l = lane < N
    seg = lax.broadcasted_iota(jnp.int32, (G, RB), 0)
    oht = jnp.where(valid_l, (bb[0] == seg).astype(jnp.float32), 0.0)
    sub = lax.broadcasted_iota(jnp.int32, (RB, 1), 0) + base
    hm = jnp.where(sub < N, h3[...], 0.0)
    pooled[...] += lax.dot_general(oht, hm, (((1,), (0,)), ((), ())),
                                   preferred_element_type=jnp.float32)
    cnt[...] += lax.dot_general(oht, jnp.ones((RB, 1), jnp.float32),
                                (((1,), (0,)), ((), ())),
                                preferred_element_type=jnp.float32)

    @pl.when(i == NB - 1)
    def _():
        pm = pooled[...] / jnp.maximum(cnt[...], 1.0)
        z = jnp.maximum(jnp.dot(pm, p1[...],
                                preferred_element_type=jnp.float32) + q1[...], 0.0)
        z = jnp.maximum(jnp.dot(z, p2[...],
                                preferred_element_type=jnp.float32) + q2[...], 0.0)
        out[...] = jnp.dot(z, p3[...],
                           preferred_element_type=jnp.float32) + q3[...]


_PK_SPEC = pl.BlockSpec((2, BR, 128), lambda i: (0, i, 0))
_PK_SHAPE = jax.ShapeDtypeStruct((2, PK, 128), jnp.float32)
_CNT_SPEC = pl.BlockSpec((NCORE, BR, 128), lambda i: (0, i, 0))


def _t1(cntp, xp, embp, w1):
    res = pl.pallas_call(
        _t1_body,
        grid=(NB2,),
        in_specs=[
            _CNT_SPEC,
            pl.BlockSpec((BR, 16), lambda i: (i, 0)),
            pl.BlockSpec((128, H), lambda i: (0, 0)),
            pl.BlockSpec((H, H), lambda i: (0, 0)),
        ],
        out_specs=[_PK_SPEC] * NPAIR,
        out_shape=[_PK_SHAPE] * NPAIR,
    )(cntp, xp, embp, w1)
    return list(res)


def _t2(acc4, g4, cntp, bp, h4=None, wbig=None):
    first = h4 is None
    last = wbig is None
    ins = list(acc4) + list(g4) + [cntp, bp]
    in_specs = [_PK_SPEC] * (2 * NPAIR) + [
        _CNT_SPEC,
        pl.BlockSpec((NQ, 128), lambda i: (0, 0)),
    ]
    if not first:
        ins += list(h4)
        in_specs += [_PK_SPEC] * NPAIR
    if not last:
        ins.append(wbig)
        in_specs.append(pl.BlockSpec((NQ * 128, NQ * 128), lambda i: (0, 0)))
    nout = NPAIR if last else 2 * NPAIR
    res = pl.pallas_call(
        functools.partial(_t2_body, first=first, last=last),
        grid=(NB2,),
        in_specs=in_specs,
        out_specs=[_PK_SPEC] * nout,
        out_shape=[_PK_SHAPE] * nout,
    )(*ins)
    return list(res[:NPAIR]) if last else (list(res[:NPAIR]),
                                           list(res[NPAIR:]))


def _t3(h3, bb, p1, q1, p2, q2, p3, q3):
    return pl.pallas_call(
        _t3_body,
        grid=(NB,),
        in_specs=[
            pl.BlockSpec((RB, H), lambda i: (i, 0)),
            pl.BlockSpec((1, 1, RB), lambda i: (i, 0, 0)),
            pl.BlockSpec((H, HH), lambda i: (0, 0)),
            pl.BlockSpec((1, HH), lambda i: (0, 0)),
            pl.BlockSpec((HH, 16), lambda i: (0, 0)),
            pl.BlockSpec((1, 16), lambda i: (0, 0)),
            pl.BlockSpec((16, 1), lambda i: (0, 0)),
            pl.BlockSpec((1, 1), lambda i: (0, 0)),
        ],
        out_specs=pl.BlockSpec((G, 1), lambda i: (0, 0)),
        out_shape=jax.ShapeDtypeStruct((G, 1), jnp.float32),
        scratch_shapes=[
            pltpu.VMEM((G, H), jnp.float32),
            pltpu.VMEM((G, 1), jnp.float32),
        ],
        compiler_params=pltpu.CompilerParams(
            dimension_semantics=("arbitrary",)),
    )(h3, bb, p1, q1, p2, q2, p3, q3)


def kernel(x, edge_index, batch, emb, W1, b1, W2, b2, W3, b3,
           P1, pb1, P2, pb2, P3, pb3):
    x32 = x.astype(jnp.int32)
    bat = batch.astype(jnp.int32)
    src = edge_index[0].astype(jnp.int32)
    dst = edge_index[1].astype(jnp.int32)

    xp = jnp.pad(x32, (0, NP - N)).reshape(PK, 16)
    bb = jnp.pad(bat, (0, NP - N)).reshape(NB, 1, RB)
    srcp = jnp.pad(src, (0, EP - E), constant_values=N)
    dstp = jnp.pad(dst, (0, EP - E), constant_values=N)
    src3 = srcp.reshape(NSUB, GA, K)
    dst3 = dstp.reshape(NSUB, GA, K)
    embp = jnp.pad(emb, ((0, 128 - NUM_FEAT), (0, 0)))
    zrow = jnp.zeros((K, HQ), jnp.float32)
    orow = jnp.ones((K, HQ), jnp.float32)
    dst4 = dstp.reshape(NCORE, NSUB, GC, K)

    def btile(b):
        return jnp.tile(b.reshape(NQ, HQ), (1, 16))

    def wbig(w):
        wr = w.reshape(NQ, HQ, NQ, HQ)
        i16 = jnp.eye(16, dtype=jnp.float32)
        wb = (i16[None, :, None, None, :, None]
              * wr[:, None, :, :, None, :])
        return wb.reshape(NQ * 128, NQ * 128)

    def lin4(p4):
        return [jnp.reshape(a, (2, NP, HQ)) for a in p4]

    def pk4(a4):
        return [jnp.reshape(a, (2, PK, 128)) for a in a4]

    cntp = jnp.reshape(_sc_count(dst4, zrow, orow), (NCORE, PK, 128))
    g4 = _t1(cntp, xp, embp, W1)

    acc4 = _agg(lin4(g4), src3, dst3, zrow)
    h4, g4 = _t2(pk4(acc4), g4, cntp, btile(b1), wbig=wbig(W2))
    acc4 = _agg(lin4(g4), src3, dst3, zrow)
    h4, g4 = _t2(pk4(acc4), g4, cntp, btile(b2), h4=h4, wbig=wbig(W3))
    acc4 = _agg(lin4(g4), src3, dst3, zrow)
    h4 = _t2(pk4(acc4), g4, cntp, btile(b3), h4=h4)
    h = jnp.concatenate(
        [jnp.reshape(hp, (2, NP, HQ))[s] for hp in h4 for s in (0, 1)],
        axis=1)

    return _t3(h, bb, P1, pb1.reshape(1, HH), P2, pb2.reshape(1, 16),
               P3, pb3.reshape(1, 1))


# speedup vs baseline: 10.4211x; 1.0214x over previous
import functools

import jax
import jax.numpy as jnp
from jax import lax
from jax.experimental import pallas as pl
from jax.experimental.pallas import tpu as pltpu
from jax.experimental.pallas import tpu_sc as plsc

N = 50000
E = 800000
NUM_FEAT = 92
H = 64
G = 256

RB = 2048
NB = -(-N // RB)
NP = NB * RB
K = 128
NSUB = 16
NCORE = 2
EP = -(-E // (NCORE * NSUB * K)) * (NCORE * NSUB * K)
GA = EP // (NSUB * K)
GC = EP // (NCORE * NSUB * K)
HH = H // 2
HQ = 8
NQ = H // HQ
GN = 128 // HQ
PK = NP // GN
BR = 64
NB2 = PK // BR
ROWS_PER_SUB = NP // NSUB

_mesh = functools.partial(
    plsc.VectorSubcoreMesh, core_axis_name="c", subcore_axis_name="s")


def _sc_pair2(ga, gb, src3, dst3, zrow):
    def body(ga_hbm, gb_hbm, src_hbm, dst_hbm, z_hbm, outa_hbm, outb_hbm,
             src_v, dst_v, buf0, buf1, zero_v, acc, sem0, sem1):
        c = lax.axis_index("c")
        s = lax.axis_index("s")
        pltpu.sync_copy(src_hbm.at[s], src_v)
        pltpu.sync_copy(dst_hbm.at[s], dst_v)
        pltpu.sync_copy(z_hbm, zero_v)

        def one_pass(g_hbm, out_hbm):
            for k in range(ROWS_PER_SUB // K):
                pltpu.sync_copy(zero_v,
                                acc.at[pl.ds(s * ROWS_PER_SUB + k * K, K)])
            plsc.subcore_barrier()

            gh = g_hbm.at[c]

            def gather(j, buf, sem):
                return pltpu.async_copy(gh.at[src_v.at[j]], buf, sem)

            def wait(j, buf, sem):
                pltpu.make_async_copy(gh.at[src_v.at[j]], buf, sem).wait()

            def scat(j, buf):
                pltpu.sync_copy(buf, acc.at[dst_v.at[j]], add=True)

            gather(0, buf0, sem0)

            def step(k, _):
                j0 = 2 * k
                gather(j0 + 1, buf1, sem1)
                wait(j0, buf0, sem0)
                scat(j0, buf0)
                gather(j0 + 2, buf0, sem0)
                wait(j0 + 1, buf1, sem1)
                scat(j0 + 1, buf1)
                return 0

            lax.fori_loop(0, GA // 2 - 1, step, 0)
            jlast = GA - 2
            gather(jlast + 1, buf1, sem1)
            wait(jlast, buf0, sem0)
            scat(jlast, buf0)
            wait(jlast + 1, buf1, sem1)
            scat(jlast + 1, buf1)

            plsc.subcore_barrier()
            pltpu.sync_copy(acc.at[pl.ds(s * ROWS_PER_SUB, ROWS_PER_SUB)],
                            out_hbm.at[c, pl.ds(s * ROWS_PER_SUB,
                                                ROWS_PER_SUB)])

        one_pass(ga_hbm, outa_hbm)
        one_pass(gb_hbm, outb_hbm)

    return pl.kernel(
        body,
        out_type=[jax.ShapeDtypeStruct((NCORE, NP, HQ), jnp.float32),
                  jax.ShapeDtypeStruct((NCORE, NP, HQ), jnp.float32)],
        mesh=_mesh(),
        compiler_params=pltpu.CompilerParams(use_tc_tiling_on_sc=False),
        scratch_types=[
            pltpu.VMEM((GA, K), jnp.int32),
            pltpu.VMEM((GA, K), jnp.int32),
            pltpu.VMEM((K, HQ), jnp.float32),
            pltpu.VMEM((K, HQ), jnp.float32),
            pltpu.VMEM((K, HQ), jnp.float32),
            pltpu.VMEM_SHARED((NP, HQ), jnp.float32),
            pltpu.SemaphoreType.DMA,
            pltpu.SemaphoreType.DMA,
        ],
    )(ga, gb, src3, dst3, zrow)


def _sc_count(dst4, zrow, orow):
    def body(dst_hbm, z_hbm, o_hbm, out_hbm, dst_v, zero_v, ones_v, acc):
        c = lax.axis_index("c")
        s = lax.axis_index("s")
        pltpu.sync_copy(dst_hbm.at[c, s], dst_v)
        pltpu.sync_copy(z_hbm, zero_v)
        pltpu.sync_copy(o_hbm, ones_v)
        for k in range(ROWS_PER_SUB // K):
            pltpu.sync_copy(zero_v, acc.at[pl.ds(s * ROWS_PER_SUB + k * K, K)])
        plsc.subcore_barrier()

        def step(j, _):
            pltpu.sync_copy(ones_v, acc.at[dst_v.at[j]], add=True)
            return 0

        lax.fori_loop(0, GC, step, 0)
        plsc.subcore_barrier()
        pltpu.sync_copy(acc.at[pl.ds(s * ROWS_PER_SUB, ROWS_PER_SUB)],
                        out_hbm.at[c, pl.ds(s * ROWS_PER_SUB, ROWS_PER_SUB)])

    return pl.kernel(
        body,
        out_type=jax.ShapeDtypeStruct((NCORE, NP, HQ), jnp.float32),
        mesh=_mesh(),
        compiler_params=pltpu.CompilerParams(use_tc_tiling_on_sc=False),
        scratch_types=[
            pltpu.VMEM((GC, K), jnp.int32),
            pltpu.VMEM((K, HQ), jnp.float32),
            pltpu.VMEM((K, HQ), jnp.float32),
            pltpu.VMEM_SHARED((NP, HQ), jnp.float32),
        ],
    )(dst4, zrow, orow)


def _agg(g4, src3, dst3, zrow):
    res = []
    for qb in range(0, NPAIR, 2):
        res.extend(_sc_pair2(g4[qb], g4[qb + 1], src3, dst3, zrow))
    return res


NPAIR = NQ // 2


def _dvp(cnt):
    return lax.rsqrt(cnt[0] + cnt[1] + 1.0)


def _t1_body(cnt, xp, embp, w1, *g_outs):
    dvp = _dvp(cnt)
    e1 = jnp.dot(embp[...], w1[...], preferred_element_type=jnp.float32)
    cols = []
    for a in range(GN):
        xa = xp[:, a:a + 1]
        oh = (xa == lax.broadcasted_iota(jnp.int32, (BR, 128), 1)
              ).astype(jnp.float32)
        ra = jnp.dot(oh, e1, preferred_element_type=jnp.float32)
        cols.append(dvp[:, HQ * a:HQ * a + 1] * ra)
    for q in range(NPAIR):
        g_outs[q][...] = jnp.stack(
            [jnp.concatenate(
                [cols[a][:, HQ * j:HQ * j + HQ] for a in range(GN)], axis=1)
             for j in (2 * q, 2 * q + 1)])


def _t2_body(*refs, first, last):
    acc4 = refs[0:NPAIR]
    g4 = refs[NPAIR:2 * NPAIR]
    cnt = refs[2 * NPAIR]
    bp = refs[2 * NPAIR + 1]
    idx = 2 * NPAIR + 2
    hp4 = None if first else refs[idx:idx + NPAIR]
    idx += 0 if first else NPAIR
    wbig = None if last else refs[idx]
    idx += 0 if last else 1
    outs = refs[idx:]
    dvp = _dvp(cnt)
    slabs = []
    for q in range(NPAIR):
        hn = []
        for p in range(2):
            j = 2 * q + p
            v = jnp.maximum(dvp * (acc4[q][p] + g4[q][p]) + bp[j:j + 1, :],
                            0.0)
            if not first:
                v = v + hp4[q][p]
            hn.append(v)
        outs[q][...] = jnp.stack(hn)
        slabs.extend(hn)
    if not last:
        h_all = jnp.concatenate(slabs, axis=1)
        gn = jnp.dot(h_all, wbig[...], preferred_element_type=jnp.float32)
        for q in range(NPAIR):
            outs[NPAIR + q][...] = jnp.stack(
                [dvp * gn[:, (2 * q) * 128:(2 * q + 1) * 128],
                 dvp * gn[:, (2 * q + 1) * 128:(2 * q + 2) * 128]])


def _t3_body(h3, bb, p1, q1, p2, q2, p3, q3, out, pooled, cnt):
    i = pl.program_id(0)

    @pl.when(i == 0)
    def _():
        pooled[...] = jnp.zeros((G, H), jnp.float32)
        cnt[...] = jnp.zeros((G, 1), jnp.float32)

    base = i * RB
    lane = lax.broadcasted_iota(jnp.int32, (1, RB), 1) + base
    valid_l = lane < N
    seg = lax.broadcasted_iota(jnp.int32, (G, RB), 0)
    oht = jnp.where(valid_l, (bb[0] == seg).astype(jnp.float32), 0.0)
    sub = lax.broadcasted_iota(jnp.int32, (RB, 1), 0) + base
    hm = jnp.where(sub < N, h3[...], 0.0)
    pooled[...] += lax.dot_general(oht, hm, (((1,), (0,)), ((), ())),
                                   preferred_element_type=jnp.float32)
    cnt[...] += lax.dot_general(oht, jnp.ones((RB, 1), jnp.float32),
                                (((1,), (0,)), ((), ())),
                                preferred_element_type=jnp.float32)

    @pl.when(i == NB - 1)
    def _():
        pm = pooled[...] / jnp.maximum(cnt[...], 1.0)
        z = jnp.maximum(jnp.dot(pm, p1[...],
                                preferred_element_type=jnp.float32) + q1[...], 0.0)
        z = jnp.maximum(jnp.dot(z, p2[...],
                                preferred_element_type=jnp.float32) + q2[...], 0.0)
        out[...] = jnp.dot(z, p3[...],
                           preferred_element_type=jnp.float32) + q3[...]


_PK_SPEC = pl.BlockSpec((2, BR, 128), lambda i: (0, i, 0))
_PK_SHAPE = jax.ShapeDtypeStruct((2, PK, 128), jnp.float32)
_CNT_SPEC = pl.BlockSpec((NCORE, BR, 128), lambda i: (0, i, 0))


def _t1(cntp, xp, embp, w1):
    res = pl.pallas_call(
        _t1_body,
        grid=(NB2,),
        in_specs=[
            _CNT_SPEC,
            pl.BlockSpec((BR, GN), lambda i: (i, 0)),
            pl.BlockSpec((128, H), lambda i: (0, 0)),
            pl.BlockSpec((H, H), lambda i: (0, 0)),
        ],
        out_specs=[_PK_SPEC] * NPAIR,
        out_shape=[_PK_SHAPE] * NPAIR,
    )(cntp, xp, embp, w1)
    return list(res)


def _t2(acc4, g4, cntp, bp, h4=None, wbig=None):
    first = h4 is None
    last = wbig is None
    ins = list(acc4) + list(g4) + [cntp, bp]
    in_specs = [_PK_SPEC] * (2 * NPAIR) + [
        _CNT_SPEC,
        pl.BlockSpec((NQ, 128), lambda i: (0, 0)),
    ]
    if not first:
        ins += list(h4)
        in_specs += [_PK_SPEC] * NPAIR
    if not last:
        ins.append(wbig)
        in_specs.append(pl.BlockSpec((NQ * 128, NQ * 128), lambda i: (0, 0)))
    nout = NPAIR if last else 2 * NPAIR
    res = pl.pallas_call(
        functools.partial(_t2_body, first=first, last=last),
        grid=(NB2,),
        in_specs=in_specs,
        out_specs=[_PK_SPEC] * nout,
        out_shape=[_PK_SHAPE] * nout,
    )(*ins)
    return list(res[:NPAIR]) if last else (list(res[:NPAIR]),
                                           list(res[NPAIR:]))


def _t3(h3, bb, p1, q1, p2, q2, p3, q3):
    return pl.pallas_call(
        _t3_body,
        grid=(NB,),
        in_specs=[
            pl.BlockSpec((RB, H), lambda i: (i, 0)),
            pl.BlockSpec((1, 1, RB), lambda i: (i, 0, 0)),
            pl.BlockSpec((H, HH), lambda i: (0, 0)),
            pl.BlockSpec((1, HH), lambda i: (0, 0)),
            pl.BlockSpec((HH, 16), lambda i: (0, 0)),
            pl.BlockSpec((1, 16), lambda i: (0, 0)),
            pl.BlockSpec((16, 1), lambda i: (0, 0)),
            pl.BlockSpec((1, 1), lambda i: (0, 0)),
        ],
        out_specs=pl.BlockSpec((G, 1), lambda i: (0, 0)),
        out_shape=jax.ShapeDtypeStruct((G, 1), jnp.float32),
        scratch_shapes=[
            pltpu.VMEM((G, H), jnp.float32),
            pltpu.VMEM((G, 1), jnp.float32),
        ],
        compiler_params=pltpu.CompilerParams(
            dimension_semantics=("arbitrary",)),
    )(h3, bb, p1, q1, p2, q2, p3, q3)


def kernel(x, edge_index, batch, emb, W1, b1, W2, b2, W3, b3,
           P1, pb1, P2, pb2, P3, pb3):
    x32 = x.astype(jnp.int32)
    bat = batch.astype(jnp.int32)
    src = edge_index[0].astype(jnp.int32)
    dst = edge_index[1].astype(jnp.int32)

    xp = jnp.pad(x32, (0, NP - N)).reshape(PK, GN)
    bb = jnp.pad(bat, (0, NP - N)).reshape(NB, 1, RB)
    srcp = jnp.pad(src, (0, EP - E), constant_values=N)
    dstp = jnp.pad(dst, (0, EP - E), constant_values=N)
    src3 = srcp.reshape(NSUB, GA, K)
    dst3 = dstp.reshape(NSUB, GA, K)
    embp = jnp.pad(emb, ((0, 128 - NUM_FEAT), (0, 0)))
    zrow = jnp.zeros((K, HQ), jnp.float32)
    orow = jnp.ones((K, HQ), jnp.float32)
    dst4 = dstp.reshape(NCORE, NSUB, GC, K)

    def btile(b):
        return jnp.tile(b.reshape(NQ, HQ), (1, GN))

    def wbig(w):
        wr = w.reshape(NQ, HQ, NQ, HQ)
        ign = jnp.eye(GN, dtype=jnp.float32)
        wb = (ign[None, :, None, None, :, None]
              * wr[:, None, :, :, None, :])
        return wb.reshape(NQ * 128, NQ * 128)

    def lin4(p4):
        return [jnp.reshape(a, (2, NP, HQ)) for a in p4]

    def pk4(a4):
        return [jnp.reshape(a, (2, PK, 128)) for a in a4]

    cntp = jnp.reshape(_sc_count(dst4, zrow, orow), (NCORE, PK, 128))
    g4 = _t1(cntp, xp, embp, W1)

    acc4 = _agg(lin4(g4), src3, dst3, zrow)
    h4, g4 = _t2(pk4(acc4), g4, cntp, btile(b1), wbig=wbig(W2))
    acc4 = _agg(lin4(g4), src3, dst3, zrow)
    h4, g4 = _t2(pk4(acc4), g4, cntp, btile(b2), h4=h4, wbig=wbig(W3))
    acc4 = _agg(lin4(g4), src3, dst3, zrow)
    h4 = _t2(pk4(acc4), g4, cntp, btile(b3), h4=h4)
    h = jnp.concatenate(
        [jnp.reshape(hp, (2, NP, HQ))[s] for hp in h4 for s in (0, 1)],
        axis=1)

    return _t3(h, bb, P1, pb1.reshape(1, HH), P2, pb2.reshape(1, 16),
               P3, pb3.reshape(1, 1))
